# baseline (speedup 1.0000x reference)
import functools

import jax
import jax.numpy as jnp
from jax import lax
from jax.experimental import pallas as pl
from jax.experimental.pallas import tpu as pltpu
from jax.experimental.pallas import tpu_sc as plsc


def kernel(features, nodes):
    V, D = features.shape
    (B,) = nodes.shape
    nodes = nodes.astype(jnp.int32)

    info = plsc.get_sparse_core_info()
    nc, ns, L = info.num_cores, info.num_subcores, info.num_lanes
    nw = nc * ns
    assert B % nw == 0 and L == 16
    b_per_w = B // nw

    chunk = 256
    n_chunks = b_per_w // chunk
    assert b_per_w % chunk == 0 and n_chunks % 2 == 0 and n_chunks >= 4
    assert chunk % 128 == 0

    mesh = plsc.VectorSubcoreMesh(core_axis_name="c", subcore_axis_name="s")

    @functools.partial(
        pl.kernel,
        mesh=mesh,
        compiler_params=pltpu.CompilerParams(needs_layout_passes=False),
        out_type=jax.ShapeDtypeStruct((D, B), features.dtype),
        scratch_types=[
            pltpu.VMEM((chunk,), jnp.int32),
            pltpu.VMEM((chunk,), jnp.int32),
            pltpu.VMEM((chunk, D), features.dtype),
            pltpu.VMEM((chunk, D), features.dtype),
            pltpu.VMEM((D, 128), features.dtype),
            pltpu.SemaphoreType.DMA,
            pltpu.SemaphoreType.DMA,
            pltpu.SemaphoreType.DMA,
        ],
    )
    def gather_kernel(table_hbm, idx_hbm, out_hbm, idx_a, idx_b, rows_a,
                      rows_b, ttile, isem, gsem_a, gsem_b):
        wid = lax.axis_index("s") * nc + lax.axis_index("c")
        base = wid * b_per_w
        lanes = lax.iota(jnp.int32, L)

        def fetch(idx_s, rows_v, gsem, g):
            off = base + g * chunk
            pltpu.async_copy(idx_hbm.at[pl.ds(off, chunk)], idx_s,
                             isem).wait()

            def issue(b, carry):
                idx16 = idx_s[pl.ds(b * 16, 16)]
                rs = [idx16[k] for k in range(16)]
                for k in range(16):
                    pltpu.async_copy(
                        table_hbm.at[0, pl.ds(rs[k], 1)],
                        rows_v.at[pl.ds(b * 16 + k, 1)],
                        gsem,
                    )
                return carry

            lax.fori_loop(0, chunk // 16, issue, 0)

        def finish(rows_v, gsem, g):
            pltpu.make_async_copy(table_hbm.at[0, pl.ds(0, chunk)], rows_v,
                                  gsem).wait()
            for t2 in range(chunk // 128):
                def tp(c, carry):
                    for l0 in range(128 // L):
                        rvec = t2 * 128 + l0 * L + lanes
                        vals = plsc.load_gather(
                            rows_v, [rvec, jnp.full((L,), 0, jnp.int32) + c])
                        ttile[c, pl.ds(l0 * L, L)] = vals
                    return carry

                lax.fori_loop(0, D, tp, 0)
                off = base + g * chunk + t2 * 128
                pltpu.sync_copy(ttile,
                                out_hbm.at[pl.ds(0, D), pl.ds(off, 128)])

        fetch(idx_a, rows_a, gsem_a, 0)

        def body(t, carry):
            g = 2 * t
            fetch(idx_b, rows_b, gsem_b, g + 1)
            finish(rows_a, gsem_a, g)
            fetch(idx_a, rows_a, gsem_a, g + 2)
            finish(rows_b, gsem_b, g + 1)
            return carry

        lax.fori_loop(0, (n_chunks - 2) // 2, body, 0)
        fetch(idx_b, rows_b, gsem_b, n_chunks - 1)
        finish(rows_a, gsem_a, n_chunks - 2)
        finish(rows_b, gsem_b, n_chunks - 1)

    out_t = gather_kernel(features.reshape(1, V, D), nodes)
    return out_t.T


# speedup vs baseline: 2.0742x; 2.0742x over previous
import functools

import jax
import jax.numpy as jnp
from jax import lax
from jax.experimental import pallas as pl
from jax.experimental.pallas import tpu as pltpu
from jax.experimental.pallas import tpu_sc as plsc


def kernel(features, nodes):
    V, D = features.shape
    (B,) = nodes.shape
    nodes = nodes.astype(jnp.int32)

    info = plsc.get_sparse_core_info()
    nc, ns = info.num_cores, info.num_subcores
    nw = nc * ns
    assert B % nw == 0
    b_per_w = B // nw

    chunk = 416
    n_chunks = b_per_w // chunk
    assert b_per_w % chunk == 0 and n_chunks % 2 == 0 and n_chunks >= 4
    assert chunk % 16 == 0

    mesh = plsc.VectorSubcoreMesh(core_axis_name="c", subcore_axis_name="s")

    @functools.partial(
        pl.kernel,
        mesh=mesh,
        out_type=jax.ShapeDtypeStruct((1, B, D), features.dtype),
        scratch_types=[
            pltpu.VMEM((chunk,), jnp.int32),
            pltpu.VMEM((chunk,), jnp.int32),
            pltpu.VMEM((chunk, D), features.dtype),
            pltpu.VMEM((chunk, D), features.dtype),
            pltpu.SemaphoreType.DMA,
            pltpu.SemaphoreType.DMA,
            pltpu.SemaphoreType.DMA,
            pltpu.SemaphoreType.DMA,
        ],
    )
    def gather_kernel(table_hbm, idx_hbm, out_hbm, idx_a, idx_b, rows_a,
                      rows_b, isem_a, isem_b, gsem_a, gsem_b):
        wid = lax.axis_index("s") * nc + lax.axis_index("c")
        base = wid * b_per_w

        def idx_start(idx_s, isem, g):
            off = base + g * chunk
            pltpu.async_copy(idx_hbm.at[pl.ds(off, chunk)], idx_s, isem)

        def idx_wait(idx_s, isem):
            pltpu.make_async_copy(idx_hbm.at[pl.ds(0, chunk)], idx_s,
                                  isem).wait()

        def issue_rows(idx_s, rows_v, gsem):
            def issue(b, carry):
                idx16 = idx_s[pl.ds(b * 16, 16)]
                rs = [idx16[k] for k in range(16)]
                for k in range(16):
                    pltpu.async_copy(
                        table_hbm.at[0, pl.ds(rs[k], 1)],
                        rows_v.at[pl.ds(b * 16 + k, 1)],
                        gsem,
                    )
                return carry

            lax.fori_loop(0, chunk // 16, issue, 0)

        def finish(rows_v, gsem, g):
            pltpu.make_async_copy(table_hbm.at[0, pl.ds(0, chunk)], rows_v,
                                  gsem).wait()
            off = base + g * chunk
            pltpu.sync_copy(rows_v, out_hbm.at[0, pl.ds(off, chunk)])

        idx_start(idx_a, isem_a, 0)
        idx_start(idx_b, isem_b, 1)
        idx_wait(idx_a, isem_a)
        issue_rows(idx_a, rows_a, gsem_a)

        def body(t, carry):
            g = 2 * t
            idx_wait(idx_b, isem_b)
            issue_rows(idx_b, rows_b, gsem_b)
            idx_start(idx_a, isem_a, g + 2)
            finish(rows_a, gsem_a, g)
            idx_wait(idx_a, isem_a)
            issue_rows(idx_a, rows_a, gsem_a)
            idx_start(idx_b, isem_b, g + 3)
            finish(rows_b, gsem_b, g + 1)
            return carry

        lax.fori_loop(0, (n_chunks - 2) // 2, body, 0)
        idx_wait(idx_b, isem_b)
        issue_rows(idx_b, rows_b, gsem_b)
        finish(rows_a, gsem_a, n_chunks - 2)
        finish(rows_b, gsem_b, n_chunks - 1)

    out3 = gather_kernel(features.reshape(1, V, D), nodes)
    return out3.reshape(B, D)


# speedup vs baseline: 2.0744x; 1.0001x over previous
import functools

import jax
import jax.numpy as jnp
from jax import lax
from jax.experimental import pallas as pl
from jax.experimental.pallas import tpu as pltpu
from jax.experimental.pallas import tpu_sc as plsc


def kernel(features, nodes):
    V, D = features.shape
    (B,) = nodes.shape
    nodes = nodes.astype(jnp.int32)

    info = plsc.get_sparse_core_info()
    nc, ns = info.num_cores, info.num_subcores
    nw = nc * ns
    assert B % nw == 0
    b_per_w = B // nw

    chunk = 208
    n_chunks = b_per_w // chunk
    assert b_per_w % chunk == 0 and n_chunks % 2 == 0 and n_chunks >= 4
    assert chunk % 16 == 0

    mesh = plsc.VectorSubcoreMesh(core_axis_name="c", subcore_axis_name="s")

    @functools.partial(
        pl.kernel,
        mesh=mesh,
        out_type=jax.ShapeDtypeStruct((1, B, D), features.dtype),
        scratch_types=[
            pltpu.VMEM((chunk,), jnp.int32),
            pltpu.VMEM((chunk,), jnp.int32),
            pltpu.VMEM((chunk, D), features.dtype),
            pltpu.VMEM((chunk, D), features.dtype),
            pltpu.SemaphoreType.DMA,
            pltpu.SemaphoreType.DMA,
            pltpu.SemaphoreType.DMA,
            pltpu.SemaphoreType.DMA,
        ],
    )
    def gather_kernel(table_hbm, idx_hbm, out_hbm, idx_a, idx_b, rows_a,
                      rows_b, isem_a, isem_b, gsem_a, gsem_b):
        wid = lax.axis_index("s") * nc + lax.axis_index("c")
        base = wid * b_per_w

        def idx_start(idx_s, isem, g):
            off = base + g * chunk
            pltpu.async_copy(idx_hbm.at[pl.ds(off, chunk)], idx_s, isem)

        def idx_wait(idx_s, isem):
            pltpu.make_async_copy(idx_hbm.at[pl.ds(0, chunk)], idx_s,
                                  isem).wait()

        def issue_rows(idx_s, rows_v, gsem):
            def issue(b, carry):
                idx16 = idx_s[pl.ds(b * 16, 16)]
                rs = [idx16[k] for k in range(16)]
                for k in range(16):
                    pltpu.async_copy(
                        table_hbm.at[0, pl.ds(rs[k], 1)],
                        rows_v.at[pl.ds(b * 16 + k, 1)],
                        gsem,
                    )
                return carry

            lax.fori_loop(0, chunk // 16, issue, 0)

        def finish(rows_v, gsem, g):
            pltpu.make_async_copy(table_hbm.at[0, pl.ds(0, chunk)], rows_v,
                                  gsem).wait()
            off = base + g * chunk
            pltpu.sync_copy(rows_v, out_hbm.at[0, pl.ds(off, chunk)])

        idx_start(idx_a, isem_a, 0)
        idx_start(idx_b, isem_b, 1)
        idx_wait(idx_a, isem_a)
        issue_rows(idx_a, rows_a, gsem_a)

        def body(t, carry):
            g = 2 * t
            idx_wait(idx_b, isem_b)
            issue_rows(idx_b, rows_b, gsem_b)
            idx_start(idx_a, isem_a, g + 2)
            finish(rows_a, gsem_a, g)
            idx_wait(idx_a, isem_a)
            issue_rows(idx_a, rows_a, gsem_a)
            idx_start(idx_b, isem_b, g + 3)
            finish(rows_b, gsem_b, g + 1)
            return carry

        lax.fori_loop(0, (n_chunks - 2) // 2, body, 0)
        idx_wait(idx_b, isem_b)
        issue_rows(idx_b, rows_b, gsem_b)
        finish(rows_a, gsem_a, n_chunks - 2)
        finish(rows_b, gsem_b, n_chunks - 1)

    out3 = gather_kernel(features.reshape(1, V, D), nodes)
    return out3.reshape(B, D)
